# baseline (speedup 1.0000x reference)
import functools

import jax
import jax.numpy as jnp
from jax import lax
from jax.experimental import pallas as pl
from jax.experimental.pallas import tpu as pltpu
from jax.experimental.pallas import tpu_sc as plsc


def kernel(x, emb_weight):
    seq_len = x.shape[1]
    dim = emb_weight.shape[1]
    info = plsc.get_sparse_core_info()
    nw = info.num_cores * info.num_subcores
    rows_per_w = seq_len // nw
    mesh = plsc.VectorSubcoreMesh(core_axis_name="c", subcore_axis_name="s")

    @functools.partial(
        pl.kernel,
        mesh=mesh,
        out_type=jax.ShapeDtypeStruct((seq_len, dim), emb_weight.dtype),
        scratch_types=[pltpu.SemaphoreType.DMA],
    )
    def body(w_hbm, out_hbm, sem):
        wid = lax.axis_index("s") * info.num_cores + lax.axis_index("c")
        base = wid * rows_per_w
        pltpu.async_copy(
            w_hbm.at[pl.ds(base, rows_per_w)],
            out_hbm.at[pl.ds(base, rows_per_w)],
            sem,
        ).wait()

    return body(emb_weight[:seq_len])


# speedup vs baseline: 22.9736x; 22.9736x over previous
import functools

import jax
import jax.numpy as jnp
from jax import lax
from jax.experimental import pallas as pl
from jax.experimental.pallas import tpu as pltpu
from jax.experimental.pallas import tpu_sc as plsc

_CHUNK = 32
_NBUF = 2


def kernel(x, emb_weight):
    seq_len = x.shape[1]
    dim = emb_weight.shape[1]
    info = plsc.get_sparse_core_info()
    nw = info.num_cores * info.num_subcores
    rows_per_w = seq_len // nw
    nch = rows_per_w // _CHUNK
    mesh = plsc.VectorSubcoreMesh(core_axis_name="c", subcore_axis_name="s")

    @functools.partial(
        pl.kernel,
        mesh=mesh,
        out_type=jax.ShapeDtypeStruct((seq_len, dim), emb_weight.dtype),
        scratch_types=[
            pltpu.VMEM((_NBUF, _CHUNK, dim), emb_weight.dtype),
            pltpu.SemaphoreType.DMA,
            pltpu.SemaphoreType.DMA,
        ],
    )
    def body(w_hbm, out_hbm, buf, insem, outsem):
        wid = lax.axis_index("s") * info.num_cores + lax.axis_index("c")
        base = wid * rows_per_w

        pending_out = [None] * _NBUF
        for i in range(nch):
            b = i % _NBUF
            if pending_out[b] is not None:
                pending_out[b].wait()
            pltpu.async_copy(
                w_hbm.at[pl.ds(base + i * _CHUNK, _CHUNK)], buf.at[b], insem
            ).wait()
            pending_out[b] = pltpu.async_copy(
                buf.at[b], out_hbm.at[pl.ds(base + i * _CHUNK, _CHUNK)], outsem
            )
        for b in range(_NBUF):
            if pending_out[b] is not None:
                pending_out[b].wait()

    return body(emb_weight[:seq_len])


# speedup vs baseline: 24.8100x; 1.0799x over previous
import functools

import jax
import jax.numpy as jnp
from jax import lax
from jax.experimental import pallas as pl
from jax.experimental.pallas import tpu as pltpu
from jax.experimental.pallas import tpu_sc as plsc

_CHUNK = 32
_NBUF = 3


def kernel(x, emb_weight):
    seq_len = x.shape[1]
    dim = emb_weight.shape[1]
    info = plsc.get_sparse_core_info()
    nw = info.num_cores * info.num_subcores
    rows_per_w = seq_len // nw
    nch = rows_per_w // _CHUNK
    mesh = plsc.VectorSubcoreMesh(core_axis_name="c", subcore_axis_name="s")

    @functools.partial(
        pl.kernel,
        mesh=mesh,
        out_type=jax.ShapeDtypeStruct((seq_len, dim), emb_weight.dtype),
        scratch_types=[
            pltpu.VMEM((_NBUF, _CHUNK, dim), emb_weight.dtype),
            pltpu.SemaphoreType.DMA,
            pltpu.SemaphoreType.DMA,
        ],
    )
    def body(w_hbm, out_hbm, buf, insem, outsem):
        wid = lax.axis_index("s") * info.num_cores + lax.axis_index("c")
        base = wid * rows_per_w

        def in_copy(i):
            return pltpu.async_copy(
                w_hbm.at[pl.ds(base + i * _CHUNK, _CHUNK)], buf.at[i % _NBUF], insem
            )

        def out_copy(i):
            return pltpu.async_copy(
                buf.at[i % _NBUF], out_hbm.at[pl.ds(base + i * _CHUNK, _CHUNK)], outsem
            )

        pending_in = [None] * nch
        pending_out = [None] * nch
        for i in range(min(_NBUF - 1, nch)):
            pending_in[i] = in_copy(i)
        for i in range(nch):
            pending_in[i].wait()
            pending_out[i] = out_copy(i)
            nxt = i + _NBUF - 1
            if nxt < nch:
                prev_out = nxt - _NBUF
                if prev_out >= 0:
                    pending_out[prev_out].wait()
                pending_in[nxt] = in_copy(nxt)
        for i in range(max(0, nch - _NBUF), nch):
            pending_out[i].wait()

    return body(emb_weight[:seq_len])
